# baseline (speedup 1.0000x reference)
import jax
import jax.numpy as jnp
from jax.experimental import pallas as pl
from jax.experimental.pallas import tpu as pltpu

_BM = 1024


def _body(seq_ref, x_ref, w1h_ref, w1l_ref, b1_ref, w2h_ref, w2l_ref, b2_ref,
          w3_ref, b3_ref, out_ref, vacc):
    b = pl.program_id(0)
    j = pl.program_id(1)
    nblk = pl.num_programs(1)
    seq = seq_ref[b]

    @pl.when(j == 0)
    def _init():
        vacc[...] = jnp.zeros_like(vacc)

    @pl.when(j * _BM < seq)
    def _compute():
        x = x_ref[0]
        f32 = jnp.float32
        h = jnp.dot(x, w1h_ref[...], preferred_element_type=f32)
        h += jnp.dot(x, w1l_ref[...], preferred_element_type=f32)
        h = jnp.maximum(h + b1_ref[...], 0.0).astype(jnp.bfloat16)
        g = jnp.dot(h, w2h_ref[...], preferred_element_type=f32)
        g += jnp.dot(h, w2l_ref[...], preferred_element_type=f32)
        g = jnp.maximum(g + b2_ref[...], 0.0)
        row = jax.lax.broadcasted_iota(jnp.int32, (_BM, 1), 0) + j * _BM
        gm = jnp.where(row < seq, g, 0.0)
        vacc[...] += jnp.sum(gm, axis=0, keepdims=True)

    @pl.when(j == nblk - 1)
    def _finish():
        out_ref[b] = (jnp.sum(vacc[...] * w3_ref[...])
                      + b3_ref[0, 0] * seq.astype(jnp.float32))


def kernel(inputs, masks, sequence_lengths, W1, b1, W2, b2, W3, b3):
    del masks
    B, M, F = inputs.shape
    H = W1.shape[1]
    nblk = M // _BM
    bf16 = jnp.bfloat16

    W1h = W1.astype(bf16)
    W1l = (W1 - W1h.astype(jnp.float32)).astype(bf16)
    W2h = W2.astype(bf16)
    W2l = (W2 - W2h.astype(jnp.float32)).astype(bf16)

    def x_map(b, j, seq):
        last = (seq[b] - 1) // _BM
        return (b, jnp.minimum(j, last), 0)

    def w_map(b, j, seq):
        return (0, 0)

    grid_spec = pltpu.PrefetchScalarGridSpec(
        num_scalar_prefetch=1,
        grid=(B, nblk),
        in_specs=[
            pl.BlockSpec((1, _BM, F), x_map),
            pl.BlockSpec((F, H), w_map),
            pl.BlockSpec((F, H), w_map),
            pl.BlockSpec((1, H), w_map),
            pl.BlockSpec((H, H), w_map),
            pl.BlockSpec((H, H), w_map),
            pl.BlockSpec((1, H), w_map),
            pl.BlockSpec((1, H), w_map),
            pl.BlockSpec(memory_space=pltpu.SMEM),
        ],
        out_specs=pl.BlockSpec(memory_space=pltpu.SMEM),
        scratch_shapes=[pltpu.VMEM((1, H), jnp.float32)],
    )

    out = pl.pallas_call(
        _body,
        grid_spec=grid_spec,
        out_shape=jax.ShapeDtypeStruct((B,), jnp.float32),
    )(sequence_lengths, inputs.astype(bf16), W1h, W1l, b1.reshape(1, H),
      W2h, W2l, b2.reshape(1, H), W3.reshape(1, H), b3.reshape(1, 1))
    return out


# speedup vs baseline: 1.0008x; 1.0008x over previous
import jax
import jax.numpy as jnp
from jax.experimental import pallas as pl
from jax.experimental.pallas import tpu as pltpu

_BM = 1024


def _body(seq_ref, x_ref, w1_ref, b1_ref, w2_ref, b2_ref,
          w3_ref, b3_ref, out_ref, vacc):
    b = pl.program_id(0)
    j = pl.program_id(1)
    nblk = pl.num_programs(1)
    seq = seq_ref[b]

    @pl.when(j == 0)
    def _init():
        vacc[...] = jnp.zeros_like(vacc)

    @pl.when(j * _BM < seq)
    def _compute():
        x = x_ref[0]
        f32 = jnp.float32
        xx = jnp.concatenate([x, x], axis=1)
        h = jnp.dot(xx, w1_ref[...], preferred_element_type=f32)
        h = jnp.maximum(h + b1_ref[...], 0.0).astype(jnp.bfloat16)
        hh = jnp.concatenate([h, h], axis=1)
        g = jnp.dot(hh, w2_ref[...], preferred_element_type=f32)
        g = jnp.maximum(g + b2_ref[...], 0.0)
        row = jax.lax.broadcasted_iota(jnp.int32, (_BM, 1), 0) + j * _BM
        gm = jnp.where(row < seq, g, 0.0)
        vacc[...] += jnp.sum(gm, axis=0, keepdims=True)

    @pl.when(j == nblk - 1)
    def _finish():
        out_ref[b] = (jnp.sum(vacc[...] * w3_ref[...])
                      + b3_ref[0, 0] * seq.astype(jnp.float32))


def kernel(inputs, masks, sequence_lengths, W1, b1, W2, b2, W3, b3):
    del masks
    B, M, F = inputs.shape
    H = W1.shape[1]
    nblk = M // _BM
    bf16 = jnp.bfloat16

    W1h = W1.astype(bf16)
    W1s = jnp.concatenate(
        [W1h, (W1 - W1h.astype(jnp.float32)).astype(bf16)], axis=0)
    W2h = W2.astype(bf16)
    W2s = jnp.concatenate(
        [W2h, (W2 - W2h.astype(jnp.float32)).astype(bf16)], axis=0)

    def x_map(b, j, seq):
        last = (seq[b] - 1) // _BM
        return (b, jnp.minimum(j, last), 0)

    def w_map(b, j, seq):
        return (0, 0)

    grid_spec = pltpu.PrefetchScalarGridSpec(
        num_scalar_prefetch=1,
        grid=(B, nblk),
        in_specs=[
            pl.BlockSpec((1, _BM, F), x_map),
            pl.BlockSpec((2 * F, H), w_map),
            pl.BlockSpec((1, H), w_map),
            pl.BlockSpec((2 * H, H), w_map),
            pl.BlockSpec((1, H), w_map),
            pl.BlockSpec((1, H), w_map),
            pl.BlockSpec(memory_space=pltpu.SMEM),
        ],
        out_specs=pl.BlockSpec(memory_space=pltpu.SMEM),
        scratch_shapes=[pltpu.VMEM((1, H), jnp.float32)],
    )

    out = pl.pallas_call(
        _body,
        grid_spec=grid_spec,
        out_shape=jax.ShapeDtypeStruct((B,), jnp.float32),
    )(sequence_lengths, inputs.astype(bf16), W1s, b1.reshape(1, H),
      W2s, b2.reshape(1, H), W3.reshape(1, H), b3.reshape(1, 1))
    return out


# speedup vs baseline: 1.8984x; 1.8970x over previous
import jax
import jax.numpy as jnp
from jax.experimental import pallas as pl
from jax.experimental.pallas import tpu as pltpu

_BM = 1024
_SUB = 512


def _body(seq_ref, x_ref, w1_ref, b1_ref, w2_ref, b2_ref,
          w3_ref, b3_ref, out_ref, vacc):
    b = pl.program_id(0)
    j = pl.program_id(1)
    nblk = pl.num_programs(1)
    seq = seq_ref[b]

    @pl.when(j == 0)
    def _init():
        vacc[...] = jnp.zeros_like(vacc)

    @pl.when(j * _BM < seq)
    def _compute():
        acc = jnp.zeros((1, vacc.shape[1]), jnp.float32)
        for s in range(_BM // _SUB):
            x = x_ref[0, s * _SUB:(s + 1) * _SUB, :]
            h = jnp.maximum(
                jnp.dot(x, w1_ref[...], preferred_element_type=jnp.float32)
                + b1_ref[...], 0.0)
            g = jnp.maximum(
                jnp.dot(h, w2_ref[...], preferred_element_type=jnp.float32)
                + b2_ref[...], 0.0)
            row = (jax.lax.broadcasted_iota(jnp.int32, (_SUB, 1), 0)
                   + j * _BM + s * _SUB)
            gm = jnp.where(row < seq, g, 0.0)
            acc += jnp.sum(gm, axis=0, keepdims=True)
        vacc[...] += acc

    @pl.when(j == nblk - 1)
    def _finish():
        out_ref[b] = (jnp.sum(vacc[...] * w3_ref[...])
                      + b3_ref[0, 0] * seq.astype(jnp.float32))


def kernel(inputs, masks, sequence_lengths, W1, b1, W2, b2, W3, b3):
    del masks
    B, M, F = inputs.shape
    H = W1.shape[1]
    nblk = M // _BM

    def x_map(b, j, seq):
        last = (seq[b] - 1) // _BM
        return (b, jnp.minimum(j, last), 0)

    def w_map(b, j, seq):
        return (0, 0)

    grid_spec = pltpu.PrefetchScalarGridSpec(
        num_scalar_prefetch=1,
        grid=(B, nblk),
        in_specs=[
            pl.BlockSpec((1, _BM, F), x_map),
            pl.BlockSpec((F, H), w_map),
            pl.BlockSpec((1, H), w_map),
            pl.BlockSpec((H, H), w_map),
            pl.BlockSpec((1, H), w_map),
            pl.BlockSpec((1, H), w_map),
            pl.BlockSpec(memory_space=pltpu.SMEM),
        ],
        out_specs=pl.BlockSpec(memory_space=pltpu.SMEM),
        scratch_shapes=[pltpu.VMEM((1, H), jnp.float32)],
    )

    out = pl.pallas_call(
        _body,
        grid_spec=grid_spec,
        out_shape=jax.ShapeDtypeStruct((B,), jnp.float32),
    )(sequence_lengths, inputs, W1, b1.reshape(1, H),
      W2, b2.reshape(1, H), W3.reshape(1, H), b3.reshape(1, 1))
    return out
